# Initial kernel scaffold; baseline (speedup 1.0000x reference)
import jax
import jax.numpy as jnp
from jax.experimental import pallas as pl


def kernel(go, emb_weights):
    raise NotImplementedError("write your pallas kernel here")


# speedup vs baseline: 1.0232x; 1.0232x over previous
import functools

import jax
import jax.numpy as jnp
from jax import lax
from jax.experimental import pallas as pl
from jax.experimental.pallas import tpu as pltpu
from jax.experimental.pallas import tpu_sc as plsc

NUM_CORES = 2
NUM_SUBCORES = 16
NUM_WORKERS = NUM_CORES * NUM_SUBCORES
CHUNK = 128


@functools.partial(jax.jit, static_argnames=())
def _gather_rows(table, idx):
    n = idx.shape[0]
    d = table.shape[1]
    per_w = n // NUM_WORKERS
    n_chunks = per_w // CHUNK
    assert per_w * NUM_WORKERS == n and n_chunks * CHUNK == per_w

    mesh = plsc.VectorSubcoreMesh(core_axis_name="c", subcore_axis_name="s")

    @functools.partial(
        pl.kernel,
        out_type=jax.ShapeDtypeStruct((n, d), jnp.float32),
        mesh=mesh,
        scratch_types=[
            pltpu.VMEM((per_w,), jnp.int32),
            pltpu.VMEM((CHUNK, d), jnp.float32),
            pltpu.SemaphoreType.DMA,
        ],
        compiler_params=pltpu.CompilerParams(use_tc_tiling_on_sc=False),
    )
    def body(table_hbm, idx_hbm, out_hbm, idx_v, rows_v, gsem):
        wid = lax.axis_index("s") * NUM_CORES + lax.axis_index("c")
        base = wid * per_w
        pltpu.sync_copy(idx_hbm.at[pl.ds(base, per_w)], idx_v)

        def chunk_body(c, carry):
            off = pl.multiple_of(c * CHUNK, CHUNK)
            pltpu.async_copy(
                table_hbm.at[idx_v.at[pl.ds(off, CHUNK)]], rows_v, gsem
            ).wait()
            pltpu.sync_copy(rows_v, out_hbm.at[pl.ds(base + off, CHUNK)])
            return carry

        lax.fori_loop(0, n_chunks, chunk_body, 0)

    return body(table, idx)


def kernel(go, emb_weights):
    b, h = go.shape
    idx = go.reshape(-1).astype(jnp.int32)
    out = _gather_rows(emb_weights, idx)
    return out.reshape(b, h, emb_weights.shape[1])


# speedup vs baseline: 1.1127x; 1.0875x over previous
import functools

import jax
import jax.numpy as jnp
from jax import lax
from jax.experimental import pallas as pl
from jax.experimental.pallas import tpu as pltpu
from jax.experimental.pallas import tpu_sc as plsc

NUM_CORES = 2
NUM_SUBCORES = 16
NUM_WORKERS = NUM_CORES * NUM_SUBCORES
CHUNK = 128
K = 4


def _gather_rows(table, idx):
    n = idx.shape[0]
    d = table.shape[1]
    per_w = n // NUM_WORKERS
    group = K * CHUNK
    n_groups = per_w // group
    n_pairs = n_groups // 2
    assert per_w * NUM_WORKERS == n
    assert n_pairs * 2 * group == per_w

    mesh = plsc.VectorSubcoreMesh(core_axis_name="c", subcore_axis_name="s")

    @functools.partial(
        pl.kernel,
        out_type=jax.ShapeDtypeStruct((n, d), jnp.float32),
        mesh=mesh,
        scratch_types=[
            pltpu.VMEM((per_w,), jnp.int32),
            pltpu.VMEM((2, group, d), jnp.float32),
            pltpu.SemaphoreType.DMA,
            pltpu.SemaphoreType.DMA,
            pltpu.SemaphoreType.DMA,
        ],
        compiler_params=pltpu.CompilerParams(use_tc_tiling_on_sc=False),
    )
    def body(table_hbm, idx_hbm, out_hbm, idx_v, rows_v, gsem0, gsem1, osem):
        wid = lax.axis_index("s") * NUM_CORES + lax.axis_index("c")
        base = wid * per_w
        pltpu.sync_copy(idx_hbm.at[pl.ds(base, per_w)], idx_v)

        def gather_desc(goff, j, half):
            return pltpu.make_async_copy(
                table_hbm.at[idx_v.at[pl.ds(goff + j * CHUNK, CHUNK)]],
                rows_v.at[half, pl.ds(j * CHUNK, CHUNK)],
                gsem0 if half == 0 else gsem1,
            )

        def issue_group(goff, half):
            for j in range(K):
                gather_desc(goff, j, half).start()

        def drain_group(half):
            for j in range(K):
                gather_desc(0, j, half).wait()

        def out_desc(goff, half):
            return pltpu.make_async_copy(
                rows_v.at[half],
                out_hbm.at[pl.ds(base + goff, group)],
                osem,
            )

        issue_group(0, 0)

        def loop_body(p, carry):
            for h in range(2):
                g = 2 * p + h
                goff = g * group
                nxt = 1 - h

                @pl.when(g + 1 < n_groups)
                def _():
                    @pl.when(g >= 1)
                    def _():
                        out_desc(0, nxt).wait()
                    issue_group(goff + group, nxt)

                drain_group(h)
                out_desc(goff, h).start()
            return carry

        lax.fori_loop(0, n_pairs, loop_body, 0)
        out_desc(0, 0).wait()
        out_desc(0, 1).wait()

    return body(table, idx)


def kernel(go, emb_weights):
    b, h = go.shape
    idx = go.reshape(-1).astype(jnp.int32)
    out = _gather_rows(emb_weights, idx)
    return out.reshape(b, h, emb_weights.shape[1])
